# Initial kernel scaffold; baseline (speedup 1.0000x reference)
import jax
import jax.numpy as jnp
from jax.experimental import pallas as pl


def kernel(node_specie, embeddings):
    raise NotImplementedError("write your pallas kernel here")


# speedup vs baseline: 1.9713x; 1.9713x over previous
import jax
import jax.numpy as jnp
from jax import lax
from jax.experimental import pallas as pl
from jax.experimental.pallas import tpu as pltpu
from jax.experimental.pallas import tpu_sc as plsc

_N_ROWS = 100000
_DIM = 128
_SCALE = 1.0 / (128.0 ** 0.5)
_C = 128
_NW = 32
_G = -(-_N_ROWS // _C)
_P = -(-_G // _NW)
_LAST_OFF = _N_ROWS - _C


def _scale_body(t_ref, o_ref):
    o_ref[...] = t_ref[...] * _SCALE


def _scaled_table(emb):
    return pl.pallas_call(
        _scale_body,
        out_shape=jax.ShapeDtypeStruct((_DIM, _DIM), jnp.float32),
    )(emb)


def _gather_body(table_hbm, idx_hbm, out_hbm, idx_v, rows_v, sem):
    w = lax.axis_index("s") * 2 + lax.axis_index("c")

    def chunk(i, carry):
        g = w * _P + i
        o = jnp.minimum(g * _C, _LAST_OFF)
        o = pl.multiple_of(o, 8)
        pltpu.sync_copy(idx_hbm.at[pl.ds(o, _C)], idx_v)
        pltpu.async_copy(table_hbm.at[idx_v], rows_v, sem).wait()
        pltpu.sync_copy(rows_v, out_hbm.at[pl.ds(o, _C)])
        return carry

    lax.fori_loop(0, _P, chunk, 0)


def kernel(node_specie, embeddings):
    idx = node_specie.astype(jnp.int32)
    w = _scaled_table(embeddings)
    mesh = plsc.VectorSubcoreMesh(core_axis_name="c", subcore_axis_name="s")
    f = pl.kernel(
        _gather_body,
        mesh=mesh,
        out_type=jax.ShapeDtypeStruct((_N_ROWS, _DIM), jnp.float32),
        scratch_types=[
            pltpu.VMEM((_C,), jnp.int32),
            pltpu.VMEM((_C, _DIM), jnp.float32),
            pltpu.SemaphoreType.DMA,
        ],
    )
    return f(w, idx)


# speedup vs baseline: 1.9853x; 1.0071x over previous
import jax
import jax.numpy as jnp
from jax import lax
from jax.experimental import pallas as pl
from jax.experimental.pallas import tpu as pltpu
from jax.experimental.pallas import tpu_sc as plsc

_N_ROWS = 100000
_DIM = 128
_SCALE = 1.0 / (128.0 ** 0.5)
_C = 128
_NW = 32
_PW = 25
_WSPAN = _PW * _C
_WLAST = _N_ROWS - _WSPAN
_NBUF = 4
_NITER = (_PW - 1) // _NBUF


def _scale_body(t_ref, o_ref):
    o_ref[...] = t_ref[...] * _SCALE


def _scaled_table(emb):
    return pl.pallas_call(
        _scale_body,
        out_shape=jax.ShapeDtypeStruct((_DIM, _DIM), jnp.float32),
    )(emb)


def _gather_body(table_hbm, idx_hbm, out_hbm, idx_v, rows_v,
                 sg0, sg1, sg2, sg3, so0, so1, so2, so3):
    sg = (sg0, sg1, sg2, sg3)
    so = (so0, so1, so2, so3)
    w = lax.axis_index("s") * 2 + lax.axis_index("c")
    base = jnp.minimum(w * _WSPAN, _WLAST)
    base = pl.multiple_of(base, 8)

    pltpu.sync_copy(idx_hbm.at[pl.ds(base, _WSPAN)], idx_v)

    def idx_slice(i):
        ii = jnp.minimum(i, _PW - 1)
        return idx_v.at[pl.ds(pl.multiple_of(ii * _C, 8), _C)], ii

    def start_gather(i, b):
        sl, _ = idx_slice(i)
        pltpu.async_copy(table_hbm.at[sl], rows_v.at[b], sg[b])

    def wait_gather(i, b):
        sl, _ = idx_slice(i)
        pltpu.make_async_copy(table_hbm.at[sl], rows_v.at[b], sg[b]).wait()

    def out_slice(i):
        _, ii = idx_slice(i)
        off = pl.multiple_of(base + ii * _C, 8)
        return out_hbm.at[pl.ds(off, _C)]

    def start_wb(i, b):
        pltpu.async_copy(rows_v.at[b], out_slice(i), so[b])

    def wait_wb(i, b):
        pltpu.make_async_copy(rows_v.at[b], out_slice(i), so[b]).wait()

    for b in range(_NBUF):
        start_gather(jnp.int32(b), b)

    def wave(j, carry):
        for b in range(_NBUF):
            i = j * _NBUF + b
            wait_gather(i, b)
            start_wb(i, b)
        for b in range(_NBUF):
            i = j * _NBUF + b
            wait_wb(i, b)
            start_gather(i + _NBUF, b)
        return carry

    lax.fori_loop(0, _NITER, wave, 0)

    last = jnp.int32(_PW - 1)
    wait_gather(last, 0)
    start_wb(last, 0)
    for b in range(1, _NBUF):
        wait_gather(last, b)
    wait_wb(last, 0)


def kernel(node_specie, embeddings):
    idx = node_specie.astype(jnp.int32)
    w = _scaled_table(embeddings)
    mesh = plsc.VectorSubcoreMesh(core_axis_name="c", subcore_axis_name="s")
    f = pl.kernel(
        _gather_body,
        mesh=mesh,
        out_type=jax.ShapeDtypeStruct((_N_ROWS, _DIM), jnp.float32),
        scratch_types=[
            pltpu.VMEM((_WSPAN,), jnp.int32),
            pltpu.VMEM((_NBUF, _C, _DIM), jnp.float32),
        ] + [pltpu.SemaphoreType.DMA] * (2 * _NBUF),
    )
    return f(w, idx)


# speedup vs baseline: 5.4500x; 2.7452x over previous
import jax
import jax.numpy as jnp
from jax import lax
from jax.experimental import pallas as pl
from jax.experimental.pallas import tpu as pltpu
from jax.experimental.pallas import tpu_sc as plsc

_N_ROWS = 100000
_DIM = 128
_SCALE = 1.0 / (128.0 ** 0.5)
_C = 128
_NW = 32
_PW = 25
_WSPAN = _PW * _C
_WLAST = _N_ROWS - _WSPAN
_NBUF = 4
_NITER = (_PW - 1) // _NBUF


def _scale_body(t_ref, o_ref):
    o_ref[...] = t_ref[...] * _SCALE


def _scaled_table(emb):
    return pl.pallas_call(
        _scale_body,
        out_shape=jax.ShapeDtypeStruct((_DIM, _DIM), jnp.float32),
    )(emb)


def _gather_body(table_hbm, idx_hbm, out_hbm, tbl_sh, idx_v, rows_v,
                 sg0, sg1, sg2, sg3, so0, so1, so2, so3):
    sg = (sg0, sg1, sg2, sg3)
    so = (so0, so1, so2, so3)
    s = lax.axis_index("s")
    w = s * 2 + lax.axis_index("c")
    base = jnp.minimum(w * _WSPAN, _WLAST)
    base = pl.multiple_of(base, 8)

    @pl.when(s == 0)
    def _():
        pltpu.sync_copy(table_hbm, rows_v.at[0])
        pltpu.sync_copy(rows_v.at[0], tbl_sh)

    pltpu.sync_copy(idx_hbm.at[pl.ds(base, _WSPAN)], idx_v)
    plsc.subcore_barrier()

    def idx_slice(i):
        ii = jnp.minimum(i, _PW - 1)
        return idx_v.at[pl.ds(pl.multiple_of(ii * _C, 8), _C)], ii

    def start_gather(i, b):
        sl, _ = idx_slice(i)
        pltpu.async_copy(tbl_sh.at[sl], rows_v.at[b], sg[b])

    def wait_gather(i, b):
        sl, _ = idx_slice(i)
        pltpu.make_async_copy(tbl_sh.at[sl], rows_v.at[b], sg[b]).wait()

    def out_slice(i):
        _, ii = idx_slice(i)
        off = pl.multiple_of(base + ii * _C, 8)
        return out_hbm.at[pl.ds(off, _C)]

    def start_wb(i, b):
        pltpu.async_copy(rows_v.at[b], out_slice(i), so[b])

    def wait_wb(i, b):
        pltpu.make_async_copy(rows_v.at[b], out_slice(i), so[b]).wait()

    for b in range(_NBUF):
        start_gather(jnp.int32(b), b)

    def wave(j, carry):
        for b in range(_NBUF):
            i = j * _NBUF + b
            wait_gather(i, b)
            start_wb(i, b)
        for b in range(_NBUF):
            i = j * _NBUF + b
            wait_wb(i, b)
            start_gather(i + _NBUF, b)
        return carry

    lax.fori_loop(0, _NITER, wave, 0)

    last = jnp.int32(_PW - 1)
    wait_gather(last, 0)
    start_wb(last, 0)
    for b in range(1, _NBUF):
        wait_gather(last, b)
    wait_wb(last, 0)


def kernel(node_specie, embeddings):
    idx = node_specie.astype(jnp.int32)
    w = _scaled_table(embeddings)
    mesh = plsc.VectorSubcoreMesh(core_axis_name="c", subcore_axis_name="s")
    f = pl.kernel(
        _gather_body,
        mesh=mesh,
        out_type=jax.ShapeDtypeStruct((_N_ROWS, _DIM), jnp.float32),
        scratch_types=[
            pltpu.VMEM_SHARED((_DIM, _DIM), jnp.float32),
            pltpu.VMEM((_WSPAN,), jnp.int32),
            pltpu.VMEM((_NBUF, _C, _DIM), jnp.float32),
        ] + [pltpu.SemaphoreType.DMA] * (2 * _NBUF),
    )
    return f(w, idx)
